# baseline (speedup 1.0000x reference)
import jax
import jax.numpy as jnp
from jax import lax
from jax.experimental import pallas as pl
from jax.experimental.pallas import tpu as pltpu
from jax.experimental.pallas import tpu_sc as plsc

_BATCH = 16384
_FIELDS = 26
_DIM = 32
_VOCAB = 1000000
_TOTAL = _BATCH * _FIELDS
_NUM_CORES = 2
_NUM_SUBCORES = 16
_NW = _NUM_CORES * _NUM_SUBCORES
_B_PER_W = _TOTAL // _NW
_N_CHUNKS = 8
_CHUNK = _B_PER_W // _N_CHUNKS
_NBUF = 2


def _gather_body(idx_hbm, table_hbm, out_hbm, idx_v, rows0, rows1, s0, s1):
    wid = lax.axis_index("s") * _NUM_CORES + lax.axis_index("c")
    base = wid * _B_PER_W
    rows = (rows0, rows1)
    sems = (s0, s1)

    pltpu.sync_copy(idx_hbm.at[pl.ds(base, _B_PER_W)], idx_v)

    def start(c, b):
        pltpu.async_copy(table_hbm.at[idx_v.at[pl.ds(c * _CHUNK, _CHUNK)]], rows[b], sems[b])

    def finish(c, b):
        pltpu.make_async_copy(table_hbm.at[idx_v.at[pl.ds(c * _CHUNK, _CHUNK)]], rows[b], sems[b]).wait()
        pltpu.sync_copy(
            rows[b], out_hbm.at[pl.ds(base + c * _CHUNK, _CHUNK)]
        )

    for b in range(_NBUF):
        start(b, b)

    def step(i, carry):
        c0 = i * _NBUF
        for b in range(_NBUF):
            finish(c0 + b, b)
            start(c0 + b + _NBUF, b)
        return carry

    lax.fori_loop(0, (_N_CHUNKS - _NBUF) // _NBUF, step, 0)

    for b in range(_NBUF):
        finish(_N_CHUNKS - _NBUF + b, b)


def kernel(x, table):
    idx = x.reshape(_TOTAL)
    gather = pl.kernel(
        _gather_body,
        out_type=jax.ShapeDtypeStruct((_TOTAL, _DIM), jnp.bfloat16),
        mesh=plsc.VectorSubcoreMesh(core_axis_name="c", subcore_axis_name="s"),
        scratch_types=[
            pltpu.VMEM((_B_PER_W,), jnp.int32),
            pltpu.VMEM((_CHUNK, _DIM), jnp.bfloat16),
            pltpu.VMEM((_CHUNK, _DIM), jnp.bfloat16),
            pltpu.SemaphoreType.DMA,
            pltpu.SemaphoreType.DMA,
        ],
        compiler_params=pltpu.CompilerParams(use_tc_tiling_on_sc=False),
    )
    out = gather(idx, table.astype(jnp.bfloat16))
    return out.astype(jnp.float32).reshape(_BATCH, _FIELDS, _DIM)


# speedup vs baseline: 1.3945x; 1.3945x over previous
import jax
import jax.numpy as jnp
from jax import lax
from jax.experimental import pallas as pl
from jax.experimental.pallas import tpu as pltpu
from jax.experimental.pallas import tpu_sc as plsc

_BATCH = 16384
_FIELDS = 26
_DIM = 32
_VOCAB = 1000000
_TOTAL = _BATCH * _FIELDS
_NUM_CORES = 2
_NUM_SUBCORES = 16
_NW = _NUM_CORES * _NUM_SUBCORES
_B_PER_W = _TOTAL // _NW
_N_CHUNKS = 8
_CHUNK = _B_PER_W // _N_CHUNKS
_NBUF = 2


def _gather_body(idx_hbm, table_hbm, out_hbm, idx_v, rows0, rows1, s0, s1):
    wid = lax.axis_index("s") * _NUM_CORES + lax.axis_index("c")
    base = wid * _B_PER_W
    rows = (rows0, rows1)
    sems = (s0, s1)

    pltpu.sync_copy(idx_hbm.at[pl.ds(base, _B_PER_W)], idx_v)

    def start(c, b):
        pltpu.async_copy(table_hbm.at[idx_v.at[pl.ds(c * _CHUNK, _CHUNK)]], rows[b], sems[b])

    def finish(c, b):
        pltpu.make_async_copy(table_hbm.at[idx_v.at[pl.ds(c * _CHUNK, _CHUNK)]], rows[b], sems[b]).wait()
        pltpu.sync_copy(
            rows[b], out_hbm.at[pl.ds(base + c * _CHUNK, _CHUNK)]
        )

    for b in range(_NBUF):
        start(b, b)

    def step(i, carry):
        c0 = i * _NBUF
        for b in range(_NBUF):
            finish(c0 + b, b)
            start(c0 + b + _NBUF, b)
        return carry

    lax.fori_loop(0, (_N_CHUNKS - _NBUF) // _NBUF, step, 0)

    for b in range(_NBUF):
        finish(_N_CHUNKS - _NBUF + b, b)


def kernel(x, table):
    idx = x.reshape(_TOTAL)
    gather = pl.kernel(
        _gather_body,
        out_type=jax.ShapeDtypeStruct((_TOTAL, _DIM), jnp.float32),
        mesh=plsc.VectorSubcoreMesh(core_axis_name="c", subcore_axis_name="s"),
        scratch_types=[
            pltpu.VMEM((_B_PER_W,), jnp.int32),
            pltpu.VMEM((_CHUNK, _DIM), jnp.float32),
            pltpu.VMEM((_CHUNK, _DIM), jnp.float32),
            pltpu.SemaphoreType.DMA,
            pltpu.SemaphoreType.DMA,
        ],
        compiler_params=pltpu.CompilerParams(use_tc_tiling_on_sc=False),
    )
    out = gather(idx, table)
    return out.reshape(_BATCH, _FIELDS, _DIM)
